# baseline (speedup 1.0000x reference)
import functools

import jax
import jax.numpy as jnp
from jax import lax
from jax.experimental import pallas as pl
from jax.experimental.pallas import tpu as pltpu
from jax.experimental.pallas import tpu_sc as plsc

D = 64
LANES = 16
CHUNK = 100


@functools.lru_cache(maxsize=None)
def _build(n_chunks: int, vocab: int, max_seq: int):
    mesh = plsc.VectorSubcoreMesh(core_axis_name="c", subcore_axis_name="s")
    info = plsc.get_sparse_core_info()
    nc, ns = info.num_cores, info.num_subcores
    nw = nc * ns
    assert n_chunks % nw == 0
    per_w = n_chunks // nw

    @functools.partial(
        pl.kernel,
        out_type=jax.ShapeDtypeStruct((n_chunks, CHUNK, D), jnp.float32),
        mesh=mesh,
        scratch_types=[
            pltpu.VMEM((CHUNK,), jnp.int32),
            pltpu.VMEM((CHUNK, D), jnp.float32),
            pltpu.VMEM((max_seq, D), jnp.float32),
            pltpu.SemaphoreType.DMA,
        ],
        compiler_params=pltpu.CompilerParams(use_tc_tiling_on_sc=False),
    )
    def k(token_hbm, wt_hbm, pos_hbm, out_hbm, idx_v, rows_v, pos_v, sem):
        wid = lax.axis_index("s") * nc + lax.axis_index("c")
        pltpu.sync_copy(pos_hbm, pos_v)

        def chunk_body(c, carry):
            chunk = wid * per_w + c
            pltpu.sync_copy(token_hbm.at[chunk], idx_v)
            pltpu.async_copy(wt_hbm.at[idx_v], rows_v, sem).wait()
            half = (chunk % 2) * CHUNK

            def row_body(r, carry2):
                pr = half + r
                for j in range(D // LANES):
                    sl = pl.ds(j * LANES, LANES)
                    rows_v[r, sl] = rows_v[r, sl] * 8.0 + pos_v[pr, sl]
                return carry2

            lax.fori_loop(0, CHUNK, row_body, 0)
            pltpu.sync_copy(rows_v, out_hbm.at[chunk])
            return carry

        lax.fori_loop(0, per_w, chunk_body, 0)

    return k


def kernel(token, word_table, pos_table):
    b, s = token.shape
    vocab, d = word_table.shape
    max_seq = pos_table.shape[0]
    assert d == D and s % (2 * CHUNK) == 0 and max_seq == 2 * CHUNK
    token2 = token.reshape(-1, CHUNK)
    out = _build(token2.shape[0], vocab, max_seq)(token2, word_table, pos_table)
    return out.reshape(b, s, d)


# speedup vs baseline: 1.2507x; 1.2507x over previous
import functools

import jax
import jax.numpy as jnp
from jax import lax
from jax.experimental import pallas as pl
from jax.experimental.pallas import tpu as pltpu
from jax.experimental.pallas import tpu_sc as plsc

D = 64
LANES = 16
CHUNK = 100
NBUF = 4


@functools.lru_cache(maxsize=None)
def _build(n_chunks: int, vocab: int, max_seq: int):
    mesh = plsc.VectorSubcoreMesh(core_axis_name="c", subcore_axis_name="s")
    info = plsc.get_sparse_core_info()
    nc, ns = info.num_cores, info.num_subcores
    nw = nc * ns
    assert n_chunks % (nw * NBUF) == 0
    chunks_per_w = n_chunks // nw
    n_blks = chunks_per_w // NBUF

    @functools.partial(
        pl.kernel,
        out_type=jax.ShapeDtypeStruct((n_chunks, CHUNK, D), jnp.float32),
        mesh=mesh,
        scratch_types=[
            pltpu.VMEM((chunks_per_w, CHUNK), jnp.int32),
            pltpu.VMEM((NBUF, CHUNK, D), jnp.float32),
            pltpu.VMEM((max_seq, D), jnp.float32),
            pltpu.SemaphoreType.DMA,
            pltpu.SemaphoreType.DMA,
        ],
        compiler_params=pltpu.CompilerParams(use_tc_tiling_on_sc=False),
    )
    def k(token_hbm, wt_hbm, pos_hbm, out_hbm, idx_all, bufs, pos_v, gsem, wsem):
        wid = lax.axis_index("s") * nc + lax.axis_index("c")
        chunk0 = wid * chunks_per_w
        pltpu.sync_copy(pos_hbm, pos_v)
        pltpu.sync_copy(token_hbm.at[pl.ds(chunk0, chunks_per_w)], idx_all)

        def issue_gather(c, b):
            pltpu.async_copy(wt_hbm.at[idx_all.at[c]], bufs.at[b], gsem)

        def drain(sem, b):
            pltpu.make_async_copy(out_hbm.at[0], bufs.at[b], sem).wait()

        for t in range(NBUF - 1):
            issue_gather(t, t)

        def blk_body(blk, carry):
            for b in range(NBUF):
                c = blk * NBUF + b
                drain(gsem, b)
                half = (b % 2) * CHUNK

                def row_body(r, c2, _b=b, _half=half):
                    pr = _half + r
                    for j in range(D // LANES):
                        sl = pl.ds(j * LANES, LANES)
                        bufs[_b, r, sl] = bufs[_b, r, sl] * 8.0 + pos_v[pr, sl]
                    return c2

                lax.fori_loop(0, CHUNK, row_body, 0)
                pltpu.async_copy(bufs.at[b], out_hbm.at[chunk0 + c], wsem)
                nxt = c + NBUF - 1
                nb = (b + NBUF - 1) % NBUF

                @pl.when(nxt < chunks_per_w)
                def _():
                    @pl.when(c >= 1)
                    def _():
                        drain(wsem, nb)

                    issue_gather(nxt, nb)

            return carry

        lax.fori_loop(0, n_blks, blk_body, 0)

        for t in range(NBUF):
            drain(wsem, t)

    return k


def kernel(token, word_table, pos_table):
    b, s = token.shape
    vocab, d = word_table.shape
    max_seq = pos_table.shape[0]
    assert d == D and s % (2 * CHUNK) == 0 and max_seq == 2 * CHUNK
    token2 = token.reshape(-1, CHUNK)
    out = _build(token2.shape[0], vocab, max_seq)(token2, word_table, pos_table)
    return out.reshape(b, s, d)


# speedup vs baseline: 1.2531x; 1.0019x over previous
import functools

import jax
import jax.numpy as jnp
from jax import lax
from jax.experimental import layout as jlayout
from jax.experimental import pallas as pl
from jax.experimental.pallas import tpu as pltpu
from jax.experimental.pallas import tpu_sc as plsc

D = 64
LANES = 16
CHUNK = 100
NBUF = 4


@functools.lru_cache(maxsize=None)
def _build(n_chunks: int, vocab: int, max_seq: int):
    mesh = plsc.VectorSubcoreMesh(core_axis_name="c", subcore_axis_name="s")
    info = plsc.get_sparse_core_info()
    nc, ns = info.num_cores, info.num_subcores
    nw = nc * ns
    assert n_chunks % (nw * NBUF) == 0
    chunks_per_w = n_chunks // nw
    n_blks = chunks_per_w // NBUF

    n_seq = n_chunks // 2
    seq_per_w = n_seq // nw

    @functools.partial(
        pl.kernel,
        out_type=jax.ShapeDtypeStruct((n_seq, 2 * CHUNK, D), jnp.float32),
        mesh=mesh,
        scratch_types=[
            pltpu.VMEM((chunks_per_w, CHUNK), jnp.int32),
            pltpu.VMEM((NBUF, CHUNK, D), jnp.float32),
            pltpu.VMEM((max_seq, D), jnp.float32),
            pltpu.SemaphoreType.DMA,
            pltpu.SemaphoreType.DMA,
        ],
        compiler_params=pltpu.CompilerParams(use_tc_tiling_on_sc=False),
    )
    def k(token_hbm, wt_hbm, pos_hbm, out_hbm, idx_all, bufs, pos_v, gsem, wsem):
        wid = lax.axis_index("s") * nc + lax.axis_index("c")
        chunk0 = wid * chunks_per_w
        pltpu.sync_copy(pos_hbm, pos_v)
        pltpu.sync_copy(token_hbm.at[pl.ds(chunk0, chunks_per_w)], idx_all)

        def issue_gather(c, b):
            pltpu.async_copy(wt_hbm.at[idx_all.at[c]], bufs.at[b], gsem)

        def drain(sem, b):
            pltpu.make_async_copy(
                out_hbm.at[0, pl.ds(0, CHUNK)], bufs.at[b], sem
            ).wait()

        for t in range(NBUF - 1):
            issue_gather(t, t)

        def blk_body(blk, carry):
            for b in range(NBUF):
                c = blk * NBUF + b
                drain(gsem, b)
                half = (b % 2) * CHUNK

                def row_body(r, c2, _b=b, _half=half):
                    pr = _half + r
                    for j in range(D // LANES):
                        sl = pl.ds(j * LANES, LANES)
                        bufs[_b, r, sl] = bufs[_b, r, sl] * 8.0 + pos_v[pr, sl]
                    return c2

                lax.fori_loop(0, CHUNK, row_body, 0)
                seq = wid * seq_per_w + (c // 2)
                pltpu.async_copy(
                    bufs.at[b], out_hbm.at[seq, pl.ds(half, CHUNK)], wsem
                )
                nxt = c + NBUF - 1
                nb = (b + NBUF - 1) % NBUF

                @pl.when(nxt < chunks_per_w)
                def _():
                    @pl.when(c >= 1)
                    def _():
                        drain(wsem, nb)

                    issue_gather(nxt, nb)

            return carry

        lax.fori_loop(0, n_blks, blk_body, 0)

        for t in range(NBUF):
            drain(wsem, t)

    return k


def kernel(token, word_table, pos_table):
    b, s = token.shape
    vocab, d = word_table.shape
    max_seq = pos_table.shape[0]
    assert d == D and s % (2 * CHUNK) == 0 and max_seq == 2 * CHUNK
    token2 = token.reshape(-1, CHUNK)
    wt_lin = jlayout.with_layout_constraint(
        word_table, jlayout.Layout((1, 0), ())
    )
    return _build(token2.shape[0], vocab, max_seq)(token2, wt_lin, pos_table)
